# Initial kernel scaffold; baseline (speedup 1.0000x reference)
import jax
import jax.numpy as jnp
from jax.experimental import pallas as pl


def kernel(input, target, X):
    raise NotImplementedError("write your pallas kernel here")


# speedup vs baseline: 82.4480x; 82.4480x over previous
import jax
import jax.numpy as jnp
from jax.experimental import pallas as pl

_MARGIN = 0.5
_ALPHA = 0.5
_BSZ = 4096
_K = 8
_C_REAL = 1000
_CPAD = 1024
_CBLK = 128
_NBLK = _CPAD // _CBLK


def _stats_kernel(x_ref, t_ref, out_ref):
    i = pl.program_id(0)
    x = x_ref[...]
    t = t_ref[...]
    B = x.shape[0]
    sig = jax.nn.sigmoid(x)
    posmask = t == 1.0
    colid = jax.lax.broadcasted_iota(jnp.int32, x.shape, 1) + i * _CBLK
    real = colid < _C_REAL
    bce_el = jnp.maximum(x, 0.0) - x * t + jnp.log1p(jnp.exp(-jnp.abs(x)))
    bce_col = jnp.sum(jnp.where(real, bce_el, 0.0), axis=0, keepdims=True)
    counts = jnp.sum(t, axis=0, keepdims=True)
    rowiota = jax.lax.broadcasted_iota(jnp.int32, x.shape, 0)
    inf = jnp.float32(jnp.inf)
    pos_pred = jnp.where(posmask, sig, inf)
    neg_pred = jnp.where(posmask, inf, sig)
    pvals = []
    for _ in range(_K + 1):
        v = jnp.min(pos_pred, axis=0, keepdims=True)
        r = jnp.min(jnp.where(pos_pred == v, rowiota, B), axis=0, keepdims=True)
        pos_pred = jnp.where(rowiota == r, inf, pos_pred)
        pvals.append(v)
    rem_mask = pos_pred != inf
    nvals = []
    for _ in range(_K):
        v = jnp.min(neg_pred, axis=0, keepdims=True)
        r = jnp.min(jnp.where(neg_pred == v, rowiota, B), axis=0, keepdims=True)
        neg_pred = jnp.where(rowiota == r, inf, neg_pred)
        nvals.append(v)
    npos = counts
    n_n = jnp.minimum(jnp.float32(_K), jnp.float32(B) - npos)
    cp = jnp.zeros_like(counts)
    for j in range(_K + 1):
        term = jnp.sum(jnp.where(posmask, jnp.abs(sig - pvals[j]), 0.0),
                       axis=0, keepdims=True)
        cp = cp + jnp.where(jnp.float32(j) < npos, term, 0.0)
    corr = jnp.sum(jnp.where(rem_mask, jnp.abs(sig - pvals[_K]), 0.0),
                   axis=0, keepdims=True)
    cp = cp - corr
    cn = jnp.zeros_like(counts)
    for j in range(_K):
        term = jnp.sum(jnp.where(posmask, jnp.abs(sig - nvals[j]), 0.0),
                       axis=0, keepdims=True)
        cn = cn + jnp.where(jnp.float32(j) < n_n, term, 0.0)
    zero = jnp.zeros_like(counts)
    packed = jnp.concatenate(
        [counts, cp, cn, bce_col, zero, zero, zero, zero], axis=0)
    out_ref[...] = packed[None]


def _combine_kernel(cc_ref, cr_ref, cp_ref, cn_ref, bce_ref, out_ref):
    ci = cc_ref[...]
    cj = cr_ref[...]
    cp = cp_ref[...]
    cn = cn_ref[...]
    bce_cols = bce_ref[...]
    ii = jax.lax.broadcasted_iota(jnp.int32, (_CPAD, _CPAD), 0)
    jj = jax.lax.broadcasted_iota(jnp.int32, (_CPAD, _CPAD), 1)
    before = (cj < ci) | ((cj == ci) & (jj <= ii))
    csum = jnp.sum(jnp.where(before, jnp.broadcast_to(cj, (_CPAD, _CPAD)), 0.0),
                   axis=1, keepdims=True)
    sel = (csum <= 0.5 * _BSZ) & (ci > 1.0)
    n_n = jnp.minimum(jnp.float32(_K), jnp.float32(_BSZ) - ci)
    n_p = jnp.minimum(jnp.float32(_K), ci - 1.0)
    col_valid = sel & (n_n >= 1.0)
    dp = jnp.sum(jnp.where(col_valid, n_n * cp, 0.0))
    dn = jnp.sum(jnp.where(col_valid, n_p * cn, 0.0))
    bce = jnp.sum(bce_cols) / jnp.float32(_BSZ * _C_REAL)
    crl = jnp.maximum(dp - dn + _MARGIN, 0.0)
    has_trip = jnp.any(sel)
    res = jnp.where(has_trip, _ALPHA * crl + (1.0 - _ALPHA) * bce, bce)
    out_ref[...] = res.reshape(1, 1)


@jax.jit
def _impl(x, t):
    xp = jnp.pad(x, ((0, 0), (0, _CPAD - x.shape[1])))
    tp = jnp.pad(t, ((0, 0), (0, _CPAD - t.shape[1])))
    stats = pl.pallas_call(
        _stats_kernel,
        grid=(_NBLK,),
        in_specs=[pl.BlockSpec((_BSZ, _CBLK), lambda i: (0, i)),
                  pl.BlockSpec((_BSZ, _CBLK), lambda i: (0, i))],
        out_specs=pl.BlockSpec((1, 8, _CBLK), lambda i: (i, 0, 0)),
        out_shape=jax.ShapeDtypeStruct((_NBLK, 8, _CBLK), jnp.float32),
    )(xp, tp)
    counts = stats[:, 0, :].reshape(_CPAD, 1)
    cp = stats[:, 1, :].reshape(_CPAD, 1)
    cn = stats[:, 2, :].reshape(_CPAD, 1)
    bce = stats[:, 3, :].reshape(_CPAD, 1)
    counts_row = counts.reshape(1, _CPAD)
    out = pl.pallas_call(
        _combine_kernel,
        out_shape=jax.ShapeDtypeStruct((1, 1), jnp.float32),
    )(counts, counts_row, cp, cn, bce)
    return out[0, 0]


def kernel(input, target, X):
    return _impl(input, target)


# speedup vs baseline: 109.8049x; 1.3318x over previous
import jax
import jax.numpy as jnp
from jax.experimental import pallas as pl
from jax.experimental.pallas import tpu as pltpu

_MARGIN = 0.5
_ALPHA = 0.5
_BSZ = 4096
_K = 8
_C_REAL = 1000
_CPAD = 1024
_CBLK = 128
_NBLK = _CPAD // _CBLK


def _stats_kernel(x_ref, t_ref, out_ref):
    i = pl.program_id(0)
    x = x_ref[...]
    t = t_ref[...]
    B = x.shape[0]
    Bf = jnp.float32(B)
    sig = jax.nn.sigmoid(x)
    posmask = t == 1.0
    colid = jax.lax.broadcasted_iota(jnp.int32, x.shape, 1) + i * _CBLK
    real = colid < _C_REAL
    bce_el = jnp.maximum(x, 0.0) - x * t + jnp.log1p(jnp.exp(-jnp.abs(x)))
    bce_col = jnp.sum(jnp.where(real, bce_el, 0.0), axis=0, keepdims=True)
    counts = jnp.sum(t, axis=0, keepdims=True)
    npos = counts
    inf = jnp.float32(jnp.inf)
    sp = jnp.where(posmask, sig, 0.0)
    s_pos = jnp.sum(sp, axis=0, keepdims=True)
    jj16 = jax.lax.broadcasted_iota(jnp.int32, (16, _CBLK), 0)

    pos_pred = jnp.where(posmask, sig, inf)
    pv = jnp.full((16, _CBLK), inf, jnp.float32)
    fill = jnp.zeros((1, _CBLK), jnp.int32)
    for _ in range(_K + 1):
        v = jnp.min(pos_pred, axis=0, keepdims=True)
        eq = pos_pred == v
        m = jnp.sum(jnp.where(eq, 1, 0).astype(jnp.int32),
                    axis=0, keepdims=True)
        pv = jnp.where((jj16 >= fill) & (jj16 < fill + m), v, pv)
        pos_pred = jnp.where(eq, inf, pos_pred)
        fill = fill + m

    neg_pred = jnp.where(posmask, inf, sig)
    nv = jnp.full((16, _CBLK), inf, jnp.float32)
    fill = jnp.zeros((1, _CBLK), jnp.int32)
    for _ in range(_K):
        v = jnp.min(neg_pred, axis=0, keepdims=True)
        eq = neg_pred == v
        m = jnp.sum(jnp.where(eq, 1, 0).astype(jnp.int32),
                    axis=0, keepdims=True)
        nv = jnp.where((jj16 >= fill) & (jj16 < fill + m), v, nv)
        neg_pred = jnp.where(eq, inf, neg_pred)
        fill = fill + m

    p8 = jnp.zeros((1, _CBLK), jnp.float32)
    for j in range(_K):
        p8 = p8 + jnp.where(jnp.float32(j) < npos, pv[j:j + 1], 0.0)
    p9 = p8 + jnp.where(jnp.float32(_K) < npos, pv[_K:_K + 1], 0.0)
    pairsum = jnp.zeros((1, _CBLK), jnp.float32)
    for r in range(_K + 1):
        rg = jnp.float32(r) < npos
        for j in range(r + 1, _K + 1):
            g = rg & (jnp.float32(j) < npos)
            pairsum = pairsum + jnp.where(
                g, jnp.abs(pv[r:r + 1] - pv[j:j + 1]), 0.0)
    pairsum = pairsum + pairsum
    extra = jnp.where(npos > jnp.float32(_K + 1),
                      8.0 * (s_pos - p9) - (npos - 9.0) * p8, 0.0)
    cp = pairsum + extra

    n_n = jnp.minimum(jnp.float32(_K), Bf - npos)
    cn = jnp.zeros((1, _CBLK), jnp.float32)
    for j in range(_K):
        nvj = nv[j:j + 1]
        s_abs = jnp.sum(jnp.abs(sp - nvj), axis=0, keepdims=True)
        cn = cn + jnp.where(jnp.float32(j) < n_n,
                            s_abs - (Bf - npos) * nvj, 0.0)

    zero = jnp.zeros_like(counts)
    packed = jnp.concatenate(
        [counts, cp, cn, bce_col, zero, zero, zero, zero], axis=0)
    out_ref[...] = packed[None]


def _combine_kernel(cc_ref, cr_ref, cp_ref, cn_ref, bce_ref, out_ref):
    ci = cc_ref[...]
    cj = cr_ref[...]
    cp = cp_ref[...]
    cn = cn_ref[...]
    bce_cols = bce_ref[...]
    ii = jax.lax.broadcasted_iota(jnp.int32, (_CPAD, _CPAD), 0)
    jj = jax.lax.broadcasted_iota(jnp.int32, (_CPAD, _CPAD), 1)
    before = (cj < ci) | ((cj == ci) & (jj <= ii))
    csum = jnp.sum(jnp.where(before, jnp.broadcast_to(cj, (_CPAD, _CPAD)), 0.0),
                   axis=1, keepdims=True)
    sel = (csum <= 0.5 * _BSZ) & (ci > 1.0)
    n_n = jnp.minimum(jnp.float32(_K), jnp.float32(_BSZ) - ci)
    n_p = jnp.minimum(jnp.float32(_K), ci - 1.0)
    col_valid = sel & (n_n >= 1.0)
    dp = jnp.sum(jnp.where(col_valid, n_n * cp, 0.0))
    dn = jnp.sum(jnp.where(col_valid, n_p * cn, 0.0))
    bce = jnp.sum(bce_cols) / jnp.float32(_BSZ * _C_REAL)
    crl = jnp.maximum(dp - dn + _MARGIN, 0.0)
    has_trip = jnp.any(sel)
    res = jnp.where(has_trip, _ALPHA * crl + (1.0 - _ALPHA) * bce, bce)
    out_ref[...] = res.reshape(1, 1)


@jax.jit
def _impl(x, t):
    xp = jnp.pad(x, ((0, 0), (0, _CPAD - x.shape[1])))
    tp = jnp.pad(t, ((0, 0), (0, _CPAD - t.shape[1])))
    stats = pl.pallas_call(
        _stats_kernel,
        grid=(_NBLK,),
        in_specs=[pl.BlockSpec((_BSZ, _CBLK), lambda i: (0, i)),
                  pl.BlockSpec((_BSZ, _CBLK), lambda i: (0, i))],
        out_specs=pl.BlockSpec((1, 8, _CBLK), lambda i: (i, 0, 0)),
        out_shape=jax.ShapeDtypeStruct((_NBLK, 8, _CBLK), jnp.float32),
        compiler_params=pltpu.CompilerParams(
            dimension_semantics=("parallel",)),
    )(xp, tp)
    counts = stats[:, 0, :].reshape(_CPAD, 1)
    cp = stats[:, 1, :].reshape(_CPAD, 1)
    cn = stats[:, 2, :].reshape(_CPAD, 1)
    bce = stats[:, 3, :].reshape(_CPAD, 1)
    counts_row = counts.reshape(1, _CPAD)
    out = pl.pallas_call(
        _combine_kernel,
        out_shape=jax.ShapeDtypeStruct((1, 1), jnp.float32),
    )(counts, counts_row, cp, cn, bce)
    return out[0, 0]


def kernel(input, target, X):
    return _impl(input, target)


# speedup vs baseline: 148.5105x; 1.3525x over previous
import jax
import jax.numpy as jnp
from jax.experimental import pallas as pl
from jax.experimental.pallas import tpu as pltpu

_MARGIN = 0.5
_ALPHA = 0.5
_BSZ = 4096
_K = 8
_C_REAL = 1000
_CPAD = 1024
_CBLK = 128
_NBLK = _CPAD // _CBLK


def _stats_kernel(x_ref, t_ref, out_ref):
    x = x_ref[...]
    t = t_ref[...]
    B = x.shape[0]
    Bf = jnp.float32(B)
    sig = jax.nn.sigmoid(x)
    posmask = t == 1.0
    bce_el = jnp.maximum(x, 0.0) - x * t - jnp.log(jnp.maximum(sig, 1.0 - sig))
    bce_col = jnp.sum(bce_el, axis=0, keepdims=True)
    counts = jnp.sum(t, axis=0, keepdims=True)
    npos = counts
    inf = jnp.float32(jnp.inf)
    sp = jnp.where(posmask, sig, 0.0)
    s_pos = jnp.sum(sp, axis=0, keepdims=True)
    jj16 = jax.lax.broadcasted_iota(jnp.int32, (16, _CBLK), 0)

    pos_pred = jnp.where(posmask, sig, inf)
    pv = jnp.full((16, _CBLK), inf, jnp.float32)
    fill = jnp.zeros((1, _CBLK), jnp.int32)
    for _ in range(_K):
        v = jnp.min(pos_pred, axis=0, keepdims=True)
        eq = pos_pred == v
        m = jnp.sum(jnp.where(eq, 1, 0).astype(jnp.int32),
                    axis=0, keepdims=True)
        pv = jnp.where((jj16 >= fill) & (jj16 < fill + m), v, pv)
        pos_pred = jnp.where(eq, inf, pos_pred)
        fill = fill + m
    v = jnp.min(pos_pred, axis=0, keepdims=True)
    pv = jnp.where(jj16 >= fill, v, pv)

    neg_pred = jnp.where(posmask, inf, sig)
    nv = jnp.full((16, _CBLK), inf, jnp.float32)
    fill = jnp.zeros((1, _CBLK), jnp.int32)
    for _ in range(_K - 1):
        v = jnp.min(neg_pred, axis=0, keepdims=True)
        eq = neg_pred == v
        m = jnp.sum(jnp.where(eq, 1, 0).astype(jnp.int32),
                    axis=0, keepdims=True)
        nv = jnp.where((jj16 >= fill) & (jj16 < fill + m), v, nv)
        neg_pred = jnp.where(eq, inf, neg_pred)
        fill = fill + m
    v = jnp.min(neg_pred, axis=0, keepdims=True)
    nv = jnp.where(jj16 >= fill, v, nv)

    p8 = jnp.zeros((1, _CBLK), jnp.float32)
    for j in range(_K):
        p8 = p8 + jnp.where(jnp.float32(j) < npos, pv[j:j + 1], 0.0)
    p9 = p8 + jnp.where(jnp.float32(_K) < npos, pv[_K:_K + 1], 0.0)
    pairsum = jnp.zeros((1, _CBLK), jnp.float32)
    for r in range(_K + 1):
        rg = jnp.float32(r) < npos
        for j in range(r + 1, _K + 1):
            g = rg & (jnp.float32(j) < npos)
            pairsum = pairsum + jnp.where(
                g, jnp.abs(pv[r:r + 1] - pv[j:j + 1]), 0.0)
    pairsum = pairsum + pairsum
    extra = jnp.where(npos > jnp.float32(_K + 1),
                      8.0 * (s_pos - p9) - (npos - 9.0) * p8, 0.0)
    cp = pairsum + extra

    n_n = jnp.minimum(jnp.float32(_K), Bf - npos)
    cn = jnp.zeros((1, _CBLK), jnp.float32)
    for j in range(_K):
        nvj = nv[j:j + 1]
        s_abs = jnp.sum(jnp.abs(sp - nvj), axis=0, keepdims=True)
        cn = cn + jnp.where(jnp.float32(j) < n_n,
                            s_abs - (Bf - npos) * nvj, 0.0)

    zero = jnp.zeros_like(counts)
    packed = jnp.concatenate(
        [counts, cp, cn, bce_col, zero, zero, zero, zero], axis=0)
    out_ref[...] = packed[None]


def _combine_kernel(cc_ref, cr_ref, cp_ref, cn_ref, bce_ref, out_ref):
    reali = jax.lax.broadcasted_iota(jnp.int32, (_CPAD, 1), 0) < _C_REAL
    realj = jax.lax.broadcasted_iota(jnp.int32, (1, _CPAD), 1) < _C_REAL
    ci = jnp.where(reali, cc_ref[...], 0.0)
    cj = jnp.where(realj, cr_ref[...], 0.0)
    cp = cp_ref[...]
    cn = cn_ref[...]
    bce_cols = jnp.where(reali, bce_ref[...], 0.0)
    ii = jax.lax.broadcasted_iota(jnp.int32, (_CPAD, _CPAD), 0)
    jj = jax.lax.broadcasted_iota(jnp.int32, (_CPAD, _CPAD), 1)
    before = (cj < ci) | ((cj == ci) & (jj <= ii))
    csum = jnp.sum(jnp.where(before, jnp.broadcast_to(cj, (_CPAD, _CPAD)), 0.0),
                   axis=1, keepdims=True)
    sel = (csum <= 0.5 * _BSZ) & (ci > 1.0) & reali
    n_n = jnp.minimum(jnp.float32(_K), jnp.float32(_BSZ) - ci)
    n_p = jnp.minimum(jnp.float32(_K), ci - 1.0)
    col_valid = sel & (n_n >= 1.0)
    dp = jnp.sum(jnp.where(col_valid, n_n * cp, 0.0))
    dn = jnp.sum(jnp.where(col_valid, n_p * cn, 0.0))
    bce = jnp.sum(bce_cols) / jnp.float32(_BSZ * _C_REAL)
    crl = jnp.maximum(dp - dn + _MARGIN, 0.0)
    has_trip = jnp.any(sel)
    res = jnp.where(has_trip, _ALPHA * crl + (1.0 - _ALPHA) * bce, bce)
    out_ref[...] = res.reshape(1, 1)


@jax.jit
def _impl(x, t):
    stats = pl.pallas_call(
        _stats_kernel,
        grid=(_NBLK,),
        in_specs=[pl.BlockSpec((_BSZ, _CBLK), lambda i: (0, i)),
                  pl.BlockSpec((_BSZ, _CBLK), lambda i: (0, i))],
        out_specs=pl.BlockSpec((1, 8, _CBLK), lambda i: (i, 0, 0)),
        out_shape=jax.ShapeDtypeStruct((_NBLK, 8, _CBLK), jnp.float32),
        compiler_params=pltpu.CompilerParams(
            dimension_semantics=("parallel",)),
    )(x, t)
    counts = stats[:, 0, :].reshape(_CPAD, 1)
    cp = stats[:, 1, :].reshape(_CPAD, 1)
    cn = stats[:, 2, :].reshape(_CPAD, 1)
    bce = stats[:, 3, :].reshape(_CPAD, 1)
    counts_row = counts.reshape(1, _CPAD)
    out = pl.pallas_call(
        _combine_kernel,
        out_shape=jax.ShapeDtypeStruct((1, 1), jnp.float32),
    )(counts, counts_row, cp, cn, bce)
    return out[0, 0]


def kernel(input, target, X):
    return _impl(input, target)


# speedup vs baseline: 156.6441x; 1.0548x over previous
import jax
import jax.numpy as jnp
from jax.experimental import pallas as pl
from jax.experimental.pallas import tpu as pltpu

_MARGIN = 0.5
_ALPHA = 0.5
_BSZ = 4096
_K = 8
_C_REAL = 1000
_CPAD = 1024
_CBLK = 128
_NBLK = _CPAD // _CBLK


def _stats_kernel(x_ref, t_ref, out_ref):
    x = x_ref[...]
    t = t_ref[...]
    B = x.shape[0]
    Bf = jnp.float32(B)
    sig = jax.nn.sigmoid(x)
    posmask = t == 1.0
    bce_el = jnp.maximum(x, 0.0) - x * t - jnp.log(jnp.maximum(sig, 1.0 - sig))
    bce_col = jnp.sum(bce_el, axis=0, keepdims=True)
    counts = jnp.sum(t, axis=0, keepdims=True)
    npos = counts
    inf = jnp.float32(jnp.inf)
    sp = jnp.where(posmask, sig, 0.0)
    s_pos = jnp.sum(sp, axis=0, keepdims=True)
    jj16 = jax.lax.broadcasted_iota(jnp.int32, (16, _CBLK), 0)

    pos_pred = jnp.where(posmask, sig, inf)
    pv = jnp.full((16, _CBLK), inf, jnp.float32)
    fill = jnp.zeros((1, _CBLK), jnp.int32)
    for _ in range(_K):
        v = jnp.min(pos_pred, axis=0, keepdims=True)
        eq = pos_pred == v
        m = jnp.sum(jnp.where(eq, 1, 0).astype(jnp.int32),
                    axis=0, keepdims=True)
        pv = jnp.where((jj16 >= fill) & (jj16 < fill + m), v, pv)
        pos_pred = jnp.where(eq, inf, pos_pred)
        fill = fill + m
    v = jnp.min(pos_pred, axis=0, keepdims=True)
    pv = jnp.where(jj16 >= fill, v, pv)

    neg_pred = jnp.where(posmask, inf, sig)
    nv = jnp.full((16, _CBLK), inf, jnp.float32)
    fill = jnp.zeros((1, _CBLK), jnp.int32)
    for _ in range(_K - 1):
        v = jnp.min(neg_pred, axis=0, keepdims=True)
        eq = neg_pred == v
        m = jnp.sum(jnp.where(eq, 1, 0).astype(jnp.int32),
                    axis=0, keepdims=True)
        nv = jnp.where((jj16 >= fill) & (jj16 < fill + m), v, nv)
        neg_pred = jnp.where(eq, inf, neg_pred)
        fill = fill + m
    v = jnp.min(neg_pred, axis=0, keepdims=True)
    nv = jnp.where(jj16 >= fill, v, nv)

    p8 = jnp.zeros((1, _CBLK), jnp.float32)
    for j in range(_K):
        p8 = p8 + jnp.where(jnp.float32(j) < npos, pv[j:j + 1], 0.0)
    p9 = p8 + jnp.where(jnp.float32(_K) < npos, pv[_K:_K + 1], 0.0)
    pairsum = jnp.zeros((1, _CBLK), jnp.float32)
    for r in range(_K + 1):
        rg = jnp.float32(r) < npos
        for j in range(r + 1, _K + 1):
            g = rg & (jnp.float32(j) < npos)
            pairsum = pairsum + jnp.where(
                g, jnp.abs(pv[r:r + 1] - pv[j:j + 1]), 0.0)
    pairsum = pairsum + pairsum
    extra = jnp.where(npos > jnp.float32(_K + 1),
                      8.0 * (s_pos - p9) - (npos - 9.0) * p8, 0.0)
    cp = pairsum + extra

    n_n = jnp.minimum(jnp.float32(_K), Bf - npos)
    cn = jnp.zeros((1, _CBLK), jnp.float32)
    for j in range(_K):
        nvj = nv[j:j + 1]
        s_abs = jnp.sum(jnp.abs(sp - nvj), axis=0, keepdims=True)
        cn = cn + jnp.where(jnp.float32(j) < n_n,
                            s_abs - (Bf - npos) * nvj, 0.0)

    zero = jnp.zeros_like(counts)
    packed = jnp.concatenate(
        [counts, cp, cn, bce_col, zero, zero, zero, zero], axis=0)
    out_ref[...] = packed[None]


def _combine_kernel(st_ref, out_ref):
    lane = jax.lax.broadcasted_iota(jnp.int32, (1, _CBLK), 1)
    counts_rows = []
    for b in range(_NBLK):
        c = st_ref[b, 0:1, :]
        real = (lane + b * _CBLK) < _C_REAL
        counts_rows.append(jnp.where(real, c, 0.0))
    src_cols = [
        jnp.transpose(jnp.broadcast_to(counts_rows[bs], (_CBLK, _CBLK)))[:, 0:1]
        for bs in range(_NBLK)]
    sidx0 = jax.lax.broadcasted_iota(jnp.int32, (_CBLK, _CBLK), 0)
    tidx0 = jax.lax.broadcasted_iota(jnp.int32, (_CBLK, _CBLK), 1)
    dp = jnp.zeros((1, _CBLK), jnp.float32)
    dn = jnp.zeros((1, _CBLK), jnp.float32)
    bce_acc = jnp.zeros((1, _CBLK), jnp.float32)
    any_sel = jnp.zeros((1, _CBLK), jnp.bool_)
    for bt in range(_NBLK):
        tgt = counts_rows[bt]
        acc = jnp.zeros((1, _CBLK), jnp.float32)
        for bs in range(_NBLK):
            sc = src_cols[bs]
            before = (sc < tgt) | ((sc == tgt)
                                   & (sidx0 + bs * _CBLK <= tidx0 + bt * _CBLK))
            acc = acc + jnp.sum(
                jnp.where(before, jnp.broadcast_to(sc, (_CBLK, _CBLK)), 0.0),
                axis=0, keepdims=True)
        real = (lane + bt * _CBLK) < _C_REAL
        sel = (acc <= 0.5 * _BSZ) & (tgt > 1.0) & real
        n_n = jnp.minimum(jnp.float32(_K), jnp.float32(_BSZ) - tgt)
        n_p = jnp.minimum(jnp.float32(_K), tgt - 1.0)
        col_valid = sel & (n_n >= 1.0)
        dp = dp + jnp.where(col_valid, n_n * st_ref[bt, 1:2, :], 0.0)
        dn = dn + jnp.where(col_valid, n_p * st_ref[bt, 2:3, :], 0.0)
        bce_acc = bce_acc + jnp.where(real, st_ref[bt, 3:4, :], 0.0)
        any_sel = any_sel | sel
    bce = jnp.sum(bce_acc) / jnp.float32(_BSZ * _C_REAL)
    crl = jnp.maximum(jnp.sum(dp) - jnp.sum(dn) + _MARGIN, 0.0)
    has_trip = jnp.any(any_sel)
    res = jnp.where(has_trip, _ALPHA * crl + (1.0 - _ALPHA) * bce, bce)
    out_ref[...] = res.reshape(1, 1)


@jax.jit
def _impl(x, t):
    stats = pl.pallas_call(
        _stats_kernel,
        grid=(_NBLK,),
        in_specs=[pl.BlockSpec((_BSZ, _CBLK), lambda i: (0, i)),
                  pl.BlockSpec((_BSZ, _CBLK), lambda i: (0, i))],
        out_specs=pl.BlockSpec((1, 8, _CBLK), lambda i: (i, 0, 0)),
        out_shape=jax.ShapeDtypeStruct((_NBLK, 8, _CBLK), jnp.float32),
        compiler_params=pltpu.CompilerParams(
            dimension_semantics=("parallel",)),
    )(x, t)
    out = pl.pallas_call(
        _combine_kernel,
        out_shape=jax.ShapeDtypeStruct((1, 1), jnp.float32),
    )(stats)
    return out[0, 0]


def kernel(input, target, X):
    return _impl(input, target)
